# baseline (speedup 1.0000x reference)
import dataclasses
import functools

import jax
import jax.numpy as jnp
from jax import lax
from jax.experimental import pallas as pl
from jax.experimental.pallas import tpu as pltpu
from jax.experimental.pallas import tpu_sc as plsc

N_NODES = 10000
D_FEAT = 256
N_EDGES = 160000
INPUT_DIM = 10000

NUM_TILES = 16
NUM_WORKERS = 32
N_PAD = 10240
NODES_PER_TILE = N_PAD // NUM_TILES
EDGES_PER_TILE = N_EDGES // NUM_TILES
DUMP = 10200
MAGIC = 12582912.0

CHUNK_ROWS = 40
NUM_CHUNKS = N_NODES // CHUNK_ROWS
MAX_CH = -(-NUM_CHUNKS // NUM_WORKERS)
WIN_ROWS = MAX_CH * CHUNK_ROWS

_f32 = jnp.float32
_i32 = jnp.int32


def _sc_body(edge_hbm, gama_hbm, bit_hbm, fea_hbm,
             feaq_o, bs_o,
             deg_sh, pres_sh, psum_sh, scale_sh, qp_sh, qn_sh,
             idx_v, ones_v, gama_v, bit_v, deg_v, pres_v,
             scale_v, qp_v, qn_v, si_v, onesn_v, iota_v, acc_v, zi_v, zf_v,
             sc_w, qp_w, qn_w, in_a, in_b, out_a, out_b,
             sem_ia, sem_ib, sem_oa, sem_ob):
    c = lax.axis_index("c")
    s = lax.axis_index("s")
    nbase = s * NODES_PER_TILE
    wid = c * NUM_TILES + s
    on_core0 = c == 0

    iota16 = lax.iota(_i32, 16)
    one16i = jnp.ones((16,), _i32)
    zero16i = jnp.zeros((16,), _i32)

    start = lax.shift_right_logical(wid * NUM_CHUNKS, 5)
    end = lax.shift_right_logical((wid + 1) * NUM_CHUNKS, 5)
    nch = end - start
    ins = (in_a, in_b)
    outs = (out_a, out_b)
    sem_in = (sem_ia, sem_ib)
    sem_out = (sem_oa, sem_ob)

    def in_slice(i):
        return fea_hbm.at[pl.ds((start + i) * CHUNK_ROWS, CHUNK_ROWS), :]

    def out_slice(i):
        return feaq_o.at[pl.ds((start + i) * CHUNK_ROWS, CHUNK_ROWS), :]

    def start_in(i, p):
        pltpu.make_async_copy(in_slice(i), ins[p], sem_in[p]).start()

    def wait_in(p):
        pltpu.make_async_copy(in_slice(0), ins[p], sem_in[p]).wait()

    def start_out(i, p):
        pltpu.make_async_copy(outs[p], out_slice(i), sem_out[p]).start()

    def wait_out(p):
        pltpu.make_async_copy(outs[p], out_slice(0), sem_out[p]).wait()

    @pl.loop(0, NODES_PER_TILE // 16)
    def _(j):
        zi_v[pl.ds(j * 16, 16)] = zero16i

    @pl.loop(0, EDGES_PER_TILE // 16)
    def _(j):
        ones_v[pl.ds(j * 16, 16)] = one16i

    @pl.loop(0, NODES_PER_TILE // 16)
    def _(j):
        onesn_v[pl.ds(j * 16, 16)] = one16i

    iota_v[...] = iota16
    zf_v[...] = jnp.zeros((16,), _f32)

    start_in(0, 0)
    start_in(1, 1)

    pltpu.sync_copy(zi_v, deg_sh.at[pl.ds(nbase, NODES_PER_TILE)])
    pltpu.sync_copy(zi_v, pres_sh.at[pl.ds(nbase, NODES_PER_TILE)])

    @pl.when(s == 0)
    def _():
        pltpu.sync_copy(zf_v, psum_sh)

    pltpu.sync_copy(gama_hbm, gama_v.at[pl.ds(0, INPUT_DIM)])
    pltpu.sync_copy(bit_hbm, bit_v.at[pl.ds(0, INPUT_DIM)])
    for k in range((N_PAD - INPUT_DIM) // 16):
        gama_v[pl.ds(INPUT_DIM + k * 16, 16)] = jnp.ones((16,), _f32)
        bit_v[pl.ds(INPUT_DIM + k * 16, 16)] = jnp.zeros((16,), _f32)

    pltpu.sync_copy(edge_hbm.at[pl.ds(N_EDGES + s * EDGES_PER_TILE,
                                      EDGES_PER_TILE)], idx_v)

    plsc.subcore_barrier()

    pltpu.sync_copy(ones_v, deg_sh.at[idx_v], add=True)

    plsc.subcore_barrier()

    pltpu.sync_copy(deg_sh.at[pl.ds(nbase, NODES_PER_TILE)], deg_v)
    for j in range(NODES_PER_TILE // 16):
        d = deg_v[pl.ds(j * 16, 16)]
        si = jnp.clip(d, 0, INPUT_DIM - 1)
        nid = nbase + j * 16 + iota16
        si = jnp.where(nid < N_NODES, si, DUMP)
        scale = plsc.load_gather(gama_v, [si])
        b = plsc.load_gather(bit_v, [si])
        br = (b + MAGIC) - MAGIC
        e = br.astype(_i32) + 126
        pw = plsc.bitcast(lax.shift_left(e, 23), _f32)
        scale_v[pl.ds(j * 16, 16)] = scale
        qp_v[pl.ds(j * 16, 16)] = pw - 1.0
        qn_v[pl.ds(j * 16, 16)] = -pw
        si_v[pl.ds(j * 16, 16)] = si
    pltpu.sync_copy(scale_v, scale_sh.at[pl.ds(nbase, NODES_PER_TILE)])
    pltpu.sync_copy(qp_v, qp_sh.at[pl.ds(nbase, NODES_PER_TILE)])
    pltpu.sync_copy(qn_v, qn_sh.at[pl.ds(nbase, NODES_PER_TILE)])
    pltpu.sync_copy(onesn_v, pres_sh.at[si_v])

    plsc.subcore_barrier()

    @pl.when(on_core0)
    def _p3():
        pltpu.sync_copy(pres_sh.at[pl.ds(nbase, NODES_PER_TILE)], pres_v)
        acc = jnp.zeros((16,), _f32)
        for j in range(NODES_PER_TILE // 16):
            p = pres_v[pl.ds(j * 16, 16)]
            bt = bit_v[pl.ds(nbase + j * 16, 16)]
            acc = acc + p.astype(_f32) * bt
        acc_v[...] = acc
        pltpu.sync_copy(acc_v, psum_sh.at[iota_v], add=True)

    plsc.subcore_barrier()

    @pl.when(on_core0 & (s == 0))
    def _p4():
        pltpu.sync_copy(psum_sh, acc_v)
        tot = jnp.sum(acc_v[...])
        val = tot * (float(D_FEAT) / 8.0 / 1024.0)
        acc_v[...] = jnp.broadcast_to(val, (16,))
        pltpu.sync_copy(acc_v.at[pl.ds(0, 1)], bs_o)

    row0 = start * CHUNK_ROWS
    pltpu.sync_copy(scale_sh.at[pl.ds(row0, WIN_ROWS)], sc_w.at[pl.ds(0, WIN_ROWS)])
    pltpu.sync_copy(qp_sh.at[pl.ds(row0, WIN_ROWS)], qp_w.at[pl.ds(0, WIN_ROWS)])
    pltpu.sync_copy(qn_sh.at[pl.ds(row0, WIN_ROWS)], qn_w.at[pl.ds(0, WIN_ROWS)])

    def compute(i, p):
        @pl.loop(0, CHUNK_ROWS // 8)
        def _grp(g):
            w0 = i * CHUNK_ROWS + g * 8
            v_sc = sc_w[pl.ds(w0, 16)]
            v_qp = qp_w[pl.ds(w0, 16)]
            v_qn = qn_w[pl.ds(w0, 16)]
            v_inv = 1.0 / v_sc
            for j in range(8):
                r = g * 8 + j
                sc_b = jnp.broadcast_to(v_sc[j], (16,))
                inv_b = jnp.broadcast_to(v_inv[j], (16,))
                qp_b = jnp.broadcast_to(v_qp[j], (16,))
                qn_b = jnp.broadcast_to(v_qn[j], (16,))
                for k in range(D_FEAT // 16):
                    x = ins[p][r, pl.ds(k * 16, 16)]
                    q = x * inv_b
                    qc = jnp.minimum(jnp.maximum(q, qn_b), qp_b)
                    qr = (qc + MAGIC) - MAGIC
                    outs[p][r, pl.ds(k * 16, 16)] = qr * sc_b

    @pl.loop(0, MAX_CH // 2)
    def _pair(t):
        iA = 2 * t
        wait_in(0)

        @pl.when(t > 0)
        def _():
            wait_out(0)

        compute(iA, 0)

        @pl.when(iA + 2 < nch)
        def _():
            start_in(iA + 2, 0)

        start_out(iA, 0)

        iB = iA + 1

        @pl.when(iB < nch)
        def _chunk_b():
            wait_in(1)

            @pl.when(t > 0)
            def _():
                wait_out(1)

            compute(iB, 1)

            @pl.when(iB + 2 < nch)
            def _():
                start_in(iB + 2, 1)

            start_out(iB, 1)

    wait_out(0)
    wait_out(1)


_sc_mesh = plsc.VectorSubcoreMesh(core_axis_name="c", subcore_axis_name="s")

_sc_params = pltpu.CompilerParams()
if "needs_layout_passes" in pltpu.CompilerParams.__dataclass_fields__:
    _sc_params = dataclasses.replace(_sc_params, needs_layout_passes=False)

_sc_call = functools.partial(
    pl.kernel,
    compiler_params=_sc_params,
    out_type=(
        jax.ShapeDtypeStruct((N_NODES, D_FEAT), _f32),
        jax.ShapeDtypeStruct((1,), _f32),
    ),
    mesh=_sc_mesh,
    scratch_types=[
        pltpu.VMEM_SHARED((N_PAD,), _i32),
        pltpu.VMEM_SHARED((N_PAD,), _i32),
        pltpu.VMEM_SHARED((16,), _f32),
        pltpu.VMEM_SHARED((N_PAD,), _f32),
        pltpu.VMEM_SHARED((N_PAD,), _f32),
        pltpu.VMEM_SHARED((N_PAD,), _f32),
        pltpu.VMEM((EDGES_PER_TILE,), _i32),
        pltpu.VMEM((EDGES_PER_TILE,), _i32),
        pltpu.VMEM((N_PAD,), _f32),
        pltpu.VMEM((N_PAD,), _f32),
        pltpu.VMEM((NODES_PER_TILE,), _i32),
        pltpu.VMEM((NODES_PER_TILE,), _i32),
        pltpu.VMEM((NODES_PER_TILE,), _f32),
        pltpu.VMEM((NODES_PER_TILE,), _f32),
        pltpu.VMEM((NODES_PER_TILE,), _f32),
        pltpu.VMEM((NODES_PER_TILE,), _i32),
        pltpu.VMEM((NODES_PER_TILE,), _i32),
        pltpu.VMEM((16,), _i32),
        pltpu.VMEM((16,), _f32),
        pltpu.VMEM((NODES_PER_TILE,), _i32),
        pltpu.VMEM((16,), _f32),
        pltpu.VMEM((WIN_ROWS + 16,), _f32),
        pltpu.VMEM((WIN_ROWS + 16,), _f32),
        pltpu.VMEM((WIN_ROWS + 16,), _f32),
        pltpu.VMEM((CHUNK_ROWS, D_FEAT), _f32),
        pltpu.VMEM((CHUNK_ROWS, D_FEAT), _f32),
        pltpu.VMEM((CHUNK_ROWS, D_FEAT), _f32),
        pltpu.VMEM((CHUNK_ROWS, D_FEAT), _f32),
        pltpu.SemaphoreType.DMA,
        pltpu.SemaphoreType.DMA,
        pltpu.SemaphoreType.DMA,
        pltpu.SemaphoreType.DMA,
    ],
)(_sc_body)


def kernel(fea, edge_index, gama, bit):
    fea_q, bs = _sc_call(
        edge_index.reshape(-1), gama.reshape(-1), bit.reshape(-1), fea)
    return fea_q, bs.reshape(())


# speedup vs baseline: 1.0484x; 1.0484x over previous
import dataclasses
import functools

import jax
import jax.numpy as jnp
from jax import lax
from jax.experimental import pallas as pl
from jax.experimental.pallas import tpu as pltpu
from jax.experimental.pallas import tpu_sc as plsc

N_NODES = 10000
D_FEAT = 256
N_EDGES = 160000
INPUT_DIM = 10000

NUM_TILES = 16
NUM_WORKERS = 32
N_PAD = 10240
NODES_PER_TILE = N_PAD // NUM_TILES
EDGES_PER_TILE = N_EDGES // NUM_TILES
DUMP = 10200
MAGIC = 12582912.0

CHUNK_ROWS = 40
NUM_CHUNKS = N_NODES // CHUNK_ROWS
MAX_CH = -(-NUM_CHUNKS // NUM_WORKERS)
WIN_ROWS = MAX_CH * CHUNK_ROWS

_f32 = jnp.float32
_i32 = jnp.int32


def _sc_body(edge_hbm, gama_hbm, bit_hbm, fea_hbm,
             feaq_o, bs_o,
             deg_sh, pres_sh, psum_sh, scale_sh, qp_sh, qn_sh,
             idx_v, ones_v, gama_v, bit_v, deg_v, pres_v,
             scale_v, qp_v, qn_v, si_v, onesn_v, iota_v, acc_v, zi_v, zf_v,
             sc_w, qp_w, qn_w, in_a, in_b, out_a, out_b,
             sem_ia, sem_ib, sem_oa, sem_ob):
    c = lax.axis_index("c")
    s = lax.axis_index("s")
    nbase = s * NODES_PER_TILE
    wid = c * NUM_TILES + s
    on_core0 = c == 0

    iota16 = lax.iota(_i32, 16)
    one16i = jnp.ones((16,), _i32)
    zero16i = jnp.zeros((16,), _i32)

    start = lax.shift_right_logical(wid * NUM_CHUNKS, 5)
    end = lax.shift_right_logical((wid + 1) * NUM_CHUNKS, 5)
    nch = end - start
    ins = (in_a, in_b)
    outs = (out_a, out_b)
    sem_in = (sem_ia, sem_ib)
    sem_out = (sem_oa, sem_ob)

    def in_slice(i):
        return fea_hbm.at[pl.ds((start + i) * CHUNK_ROWS, CHUNK_ROWS), :]

    def out_slice(i):
        return feaq_o.at[pl.ds((start + i) * CHUNK_ROWS, CHUNK_ROWS), :]

    def start_in(i, p):
        pltpu.make_async_copy(in_slice(i), ins[p], sem_in[p]).start()

    def wait_in(p):
        pltpu.make_async_copy(in_slice(0), ins[p], sem_in[p]).wait()

    def start_out(i, p):
        pltpu.make_async_copy(outs[p], out_slice(i), sem_out[p]).start()

    def wait_out(p):
        pltpu.make_async_copy(outs[p], out_slice(0), sem_out[p]).wait()

    start_in(0, 0)
    start_in(1, 1)

    @pl.loop(0, NODES_PER_TILE // 64)
    def _(j):
        for u in range(4):
            zi_v[pl.ds(j * 64 + u * 16, 16)] = zero16i
            onesn_v[pl.ds(j * 64 + u * 16, 16)] = one16i

    @pl.loop(0, EDGES_PER_TILE // 128)
    def _(j):
        for u in range(8):
            ones_v[pl.ds(j * 128 + u * 16, 16)] = one16i
    for t in range((EDGES_PER_TILE % 128) // 16):
        ones_v[pl.ds((EDGES_PER_TILE // 128) * 128 + t * 16, 16)] = one16i

    iota_v[...] = iota16
    zf_v[...] = jnp.zeros((16,), _f32)

    pltpu.sync_copy(zi_v, deg_sh.at[pl.ds(nbase, NODES_PER_TILE)])
    pltpu.sync_copy(zi_v, pres_sh.at[pl.ds(nbase, NODES_PER_TILE)])

    @pl.when(s == 0)
    def _():
        pltpu.sync_copy(zf_v, psum_sh)

    pltpu.sync_copy(gama_hbm, gama_v.at[pl.ds(0, INPUT_DIM)])
    pltpu.sync_copy(bit_hbm, bit_v.at[pl.ds(0, INPUT_DIM)])
    for k in range((N_PAD - INPUT_DIM) // 16):
        gama_v[pl.ds(INPUT_DIM + k * 16, 16)] = jnp.ones((16,), _f32)
        bit_v[pl.ds(INPUT_DIM + k * 16, 16)] = jnp.zeros((16,), _f32)

    pltpu.sync_copy(edge_hbm.at[pl.ds(N_EDGES + s * EDGES_PER_TILE,
                                      EDGES_PER_TILE)], idx_v)

    plsc.subcore_barrier()

    pltpu.sync_copy(ones_v, deg_sh.at[idx_v], add=True)

    plsc.subcore_barrier()

    pltpu.sync_copy(deg_sh.at[pl.ds(nbase, NODES_PER_TILE)], deg_v)
    for j in range(NODES_PER_TILE // 16):
        d = deg_v[pl.ds(j * 16, 16)]
        si = jnp.clip(d, 0, INPUT_DIM - 1)
        nid = nbase + j * 16 + iota16
        si = jnp.where(nid < N_NODES, si, DUMP)
        scale = plsc.load_gather(gama_v, [si])
        b = plsc.load_gather(bit_v, [si])
        br = (b + MAGIC) - MAGIC
        e = br.astype(_i32) + 126
        pw = plsc.bitcast(lax.shift_left(e, 23), _f32)
        scale_v[pl.ds(j * 16, 16)] = scale
        qp_v[pl.ds(j * 16, 16)] = pw - 1.0
        qn_v[pl.ds(j * 16, 16)] = -pw
        si_v[pl.ds(j * 16, 16)] = si
    pltpu.sync_copy(scale_v, scale_sh.at[pl.ds(nbase, NODES_PER_TILE)])
    pltpu.sync_copy(qp_v, qp_sh.at[pl.ds(nbase, NODES_PER_TILE)])
    pltpu.sync_copy(qn_v, qn_sh.at[pl.ds(nbase, NODES_PER_TILE)])
    pltpu.sync_copy(onesn_v, pres_sh.at[si_v])

    plsc.subcore_barrier()

    @pl.when(on_core0)
    def _p3():
        pltpu.sync_copy(pres_sh.at[pl.ds(nbase, NODES_PER_TILE)], pres_v)
        acc = jnp.zeros((16,), _f32)
        for j in range(NODES_PER_TILE // 16):
            p = pres_v[pl.ds(j * 16, 16)]
            bt = bit_v[pl.ds(nbase + j * 16, 16)]
            acc = acc + p.astype(_f32) * bt
        acc_v[...] = acc
        pltpu.sync_copy(acc_v, psum_sh.at[iota_v], add=True)

    row0 = start * CHUNK_ROWS
    pltpu.sync_copy(scale_sh.at[pl.ds(row0, WIN_ROWS)],
                    sc_w.at[pl.ds(0, WIN_ROWS)])
    pltpu.sync_copy(qp_sh.at[pl.ds(row0, WIN_ROWS)],
                    qp_w.at[pl.ds(0, WIN_ROWS)])
    pltpu.sync_copy(qn_sh.at[pl.ds(row0, WIN_ROWS)],
                    qn_w.at[pl.ds(0, WIN_ROWS)])

    def compute(i, p):
        @pl.loop(0, CHUNK_ROWS // 8)
        def _grp(g):
            w0 = i * CHUNK_ROWS + g * 8
            v_sc = sc_w[pl.ds(w0, 16)]
            v_qp = qp_w[pl.ds(w0, 16)]
            v_qn = qn_w[pl.ds(w0, 16)]
            v_inv = 1.0 / v_sc
            for j in range(8):
                r = g * 8 + j
                sc_b = jnp.broadcast_to(v_sc[j], (16,))
                inv_b = jnp.broadcast_to(v_inv[j], (16,))
                qp_b = jnp.broadcast_to(v_qp[j], (16,))
                qn_b = jnp.broadcast_to(v_qn[j], (16,))
                for k in range(D_FEAT // 16):
                    x = ins[p][r, pl.ds(k * 16, 16)]
                    q = x * inv_b
                    qc = jnp.minimum(jnp.maximum(q, qn_b), qp_b)
                    qr = (qc + MAGIC) - MAGIC
                    outs[p][r, pl.ds(k * 16, 16)] = qr * sc_b

    @pl.loop(0, MAX_CH // 2)
    def _pair(t):
        iA = 2 * t
        wait_in(0)

        @pl.when(t > 0)
        def _():
            wait_out(0)

        compute(iA, 0)

        @pl.when(iA + 2 < nch)
        def _():
            start_in(iA + 2, 0)

        start_out(iA, 0)

        iB = iA + 1

        @pl.when(iB < nch)
        def _chunk_b():
            wait_in(1)

            @pl.when(t > 0)
            def _():
                wait_out(1)

            compute(iB, 1)

            @pl.when(iB + 2 < nch)
            def _():
                start_in(iB + 2, 1)

            start_out(iB, 1)

    wait_out(0)
    wait_out(1)

    plsc.subcore_barrier()

    @pl.when(on_core0 & (s == 0))
    def _p4():
        pltpu.sync_copy(psum_sh, acc_v)
        tot = jnp.sum(acc_v[...])
        val = tot * (float(D_FEAT) / 8.0 / 1024.0)
        acc_v[...] = jnp.broadcast_to(val, (16,))
        pltpu.sync_copy(acc_v.at[pl.ds(0, 1)], bs_o)


_sc_mesh = plsc.VectorSubcoreMesh(core_axis_name="c", subcore_axis_name="s")

_sc_params = pltpu.CompilerParams()
if "needs_layout_passes" in pltpu.CompilerParams.__dataclass_fields__:
    _sc_params = dataclasses.replace(_sc_params, needs_layout_passes=False)

_sc_call = functools.partial(
    pl.kernel,
    compiler_params=_sc_params,
    out_type=(
        jax.ShapeDtypeStruct((N_NODES, D_FEAT), _f32),
        jax.ShapeDtypeStruct((1,), _f32),
    ),
    mesh=_sc_mesh,
    scratch_types=[
        pltpu.VMEM_SHARED((N_PAD,), _i32),
        pltpu.VMEM_SHARED((N_PAD,), _i32),
        pltpu.VMEM_SHARED((16,), _f32),
        pltpu.VMEM_SHARED((N_PAD,), _f32),
        pltpu.VMEM_SHARED((N_PAD,), _f32),
        pltpu.VMEM_SHARED((N_PAD,), _f32),
        pltpu.VMEM((EDGES_PER_TILE,), _i32),
        pltpu.VMEM((EDGES_PER_TILE,), _i32),
        pltpu.VMEM((N_PAD,), _f32),
        pltpu.VMEM((N_PAD,), _f32),
        pltpu.VMEM((NODES_PER_TILE,), _i32),
        pltpu.VMEM((NODES_PER_TILE,), _i32),
        pltpu.VMEM((NODES_PER_TILE,), _f32),
        pltpu.VMEM((NODES_PER_TILE,), _f32),
        pltpu.VMEM((NODES_PER_TILE,), _f32),
        pltpu.VMEM((NODES_PER_TILE,), _i32),
        pltpu.VMEM((NODES_PER_TILE,), _i32),
        pltpu.VMEM((16,), _i32),
        pltpu.VMEM((16,), _f32),
        pltpu.VMEM((NODES_PER_TILE,), _i32),
        pltpu.VMEM((16,), _f32),
        pltpu.VMEM((WIN_ROWS + 16,), _f32),
        pltpu.VMEM((WIN_ROWS + 16,), _f32),
        pltpu.VMEM((WIN_ROWS + 16,), _f32),
        pltpu.VMEM((CHUNK_ROWS, D_FEAT), _f32),
        pltpu.VMEM((CHUNK_ROWS, D_FEAT), _f32),
        pltpu.VMEM((CHUNK_ROWS, D_FEAT), _f32),
        pltpu.VMEM((CHUNK_ROWS, D_FEAT), _f32),
        pltpu.SemaphoreType.DMA,
        pltpu.SemaphoreType.DMA,
        pltpu.SemaphoreType.DMA,
        pltpu.SemaphoreType.DMA,
    ],
)(_sc_body)


def kernel(fea, edge_index, gama, bit):
    fea_q, bs = _sc_call(
        edge_index.reshape(-1), gama.reshape(-1), bit.reshape(-1), fea)
    return fea_q, bs.reshape(())


# speedup vs baseline: 1.0974x; 1.0468x over previous
import dataclasses
import functools

import jax
import jax.numpy as jnp
from jax import lax
from jax.experimental import pallas as pl
from jax.experimental.pallas import tpu as pltpu
from jax.experimental.pallas import tpu_sc as plsc

N_NODES = 10000
D_FEAT = 256
N_EDGES = 160000
INPUT_DIM = 10000

NUM_TILES = 16
NUM_WORKERS = 32
N_PAD = 10240
NODES_PER_TILE = N_PAD // NUM_TILES
EDGES_PER_TILE = N_EDGES // NUM_TILES
DUMP = 10200
MAGIC = 12582912.0

CHUNK_ROWS = 40
NUM_CHUNKS = N_NODES // CHUNK_ROWS
MAX_CH = -(-NUM_CHUNKS // NUM_WORKERS)
WIN_ROWS = MAX_CH * CHUNK_ROWS

_f32 = jnp.float32
_i32 = jnp.int32


def _sc_body(edge_hbm, gama_hbm, bit_hbm, fea_hbm,
             feaq_o, bs_o,
             deg_sh, pres_sh, psum_sh, scale_sh, qp_sh, qn_sh,
             idx_v, ones_v, gama_v, bit_v, deg_v, pres_v,
             scale_v, qp_v, qn_v, si_v, onesn_v, iota_v, acc_v, zi_v, zf_v,
             sc_w, qp_w, qn_w, in_a, in_b, out_a, out_b,
             sem_ia, sem_ib, sem_oa, sem_ob, sem_s1, sem_s2, sem_s3,
             sem_s4, sem_s5):
    c = lax.axis_index("c")
    s = lax.axis_index("s")
    nbase = s * NODES_PER_TILE
    wid = c * NUM_TILES + s
    on_core0 = c == 0

    iota16 = lax.iota(_i32, 16)
    one16i = jnp.ones((16,), _i32)
    zero16i = jnp.zeros((16,), _i32)

    start = lax.shift_right_logical(wid * NUM_CHUNKS, 5)
    end = lax.shift_right_logical((wid + 1) * NUM_CHUNKS, 5)
    nch = end - start
    ins = (in_a, in_b)
    outs = (out_a, out_b)
    sem_in = (sem_ia, sem_ib)
    sem_out = (sem_oa, sem_ob)

    def in_slice(i):
        return fea_hbm.at[pl.ds((start + i) * CHUNK_ROWS, CHUNK_ROWS), :]

    def out_slice(i):
        return feaq_o.at[pl.ds((start + i) * CHUNK_ROWS, CHUNK_ROWS), :]

    def start_in(i, p):
        pltpu.make_async_copy(in_slice(i), ins[p], sem_in[p]).start()

    def wait_in(p):
        pltpu.make_async_copy(in_slice(0), ins[p], sem_in[p]).wait()

    def start_out(i, p):
        pltpu.make_async_copy(outs[p], out_slice(i), sem_out[p]).start()

    def wait_out(p):
        pltpu.make_async_copy(outs[p], out_slice(0), sem_out[p]).wait()

    d_g = pltpu.make_async_copy(gama_hbm, gama_v.at[pl.ds(0, INPUT_DIM)],
                                sem_s1)
    d_b = pltpu.make_async_copy(bit_hbm, bit_v.at[pl.ds(0, INPUT_DIM)],
                                sem_s2)
    d_e = pltpu.make_async_copy(
        edge_hbm.at[pl.ds(N_EDGES + s * EDGES_PER_TILE, EDGES_PER_TILE)],
        idx_v, sem_s3)
    d_g.start()
    d_b.start()
    d_e.start()

    start_in(0, 0)
    start_in(1, 1)

    @pl.loop(0, NODES_PER_TILE // 64)
    def _(j):
        for u in range(4):
            zi_v[pl.ds(j * 64 + u * 16, 16)] = zero16i
            onesn_v[pl.ds(j * 64 + u * 16, 16)] = one16i

    @pl.loop(0, EDGES_PER_TILE // 128)
    def _(j):
        for u in range(8):
            ones_v[pl.ds(j * 128 + u * 16, 16)] = one16i
    for t in range((EDGES_PER_TILE % 128) // 16):
        ones_v[pl.ds((EDGES_PER_TILE // 128) * 128 + t * 16, 16)] = one16i

    iota_v[...] = iota16
    zf_v[...] = jnp.zeros((16,), _f32)

    d_zd = pltpu.make_async_copy(zi_v, deg_sh.at[pl.ds(nbase, NODES_PER_TILE)],
                                 sem_s4)
    d_zp = pltpu.make_async_copy(zi_v, pres_sh.at[pl.ds(nbase, NODES_PER_TILE)],
                                 sem_s5)
    d_zd.start()
    d_zp.start()

    @pl.when(s == 0)
    def _():
        pltpu.sync_copy(zf_v, psum_sh)

    for k in range((N_PAD - INPUT_DIM) // 16):
        gama_v[pl.ds(INPUT_DIM + k * 16, 16)] = jnp.ones((16,), _f32)
        bit_v[pl.ds(INPUT_DIM + k * 16, 16)] = jnp.zeros((16,), _f32)

    d_g.wait()
    d_b.wait()
    d_e.wait()
    d_zd.wait()
    d_zp.wait()

    plsc.subcore_barrier()

    pltpu.sync_copy(ones_v, deg_sh.at[idx_v], add=True)

    plsc.subcore_barrier()

    pltpu.sync_copy(deg_sh.at[pl.ds(nbase, NODES_PER_TILE)], deg_v)
    for j in range(NODES_PER_TILE // 16):
        d = deg_v[pl.ds(j * 16, 16)]
        si = jnp.clip(d, 0, INPUT_DIM - 1)
        nid = nbase + j * 16 + iota16
        si = jnp.where(nid < N_NODES, si, DUMP)
        scale = plsc.load_gather(gama_v, [si])
        b = plsc.load_gather(bit_v, [si])
        br = (b + MAGIC) - MAGIC
        e = br.astype(_i32) + 126
        pw = plsc.bitcast(lax.shift_left(e, 23), _f32)
        scale_v[pl.ds(j * 16, 16)] = scale
        qp_v[pl.ds(j * 16, 16)] = pw - 1.0
        qn_v[pl.ds(j * 16, 16)] = -pw
        si_v[pl.ds(j * 16, 16)] = si
    d_ws = pltpu.make_async_copy(
        scale_v, scale_sh.at[pl.ds(nbase, NODES_PER_TILE)], sem_s1)
    d_wp = pltpu.make_async_copy(
        qp_v, qp_sh.at[pl.ds(nbase, NODES_PER_TILE)], sem_s2)
    d_wn = pltpu.make_async_copy(
        qn_v, qn_sh.at[pl.ds(nbase, NODES_PER_TILE)], sem_s3)
    d_ws.start()
    d_wp.start()
    d_wn.start()
    pltpu.sync_copy(onesn_v, pres_sh.at[si_v])
    d_ws.wait()
    d_wp.wait()
    d_wn.wait()

    plsc.subcore_barrier()

    row0 = start * CHUNK_ROWS
    d_rw = pltpu.make_async_copy(scale_sh.at[pl.ds(row0, WIN_ROWS)],
                                 sc_w.at[pl.ds(0, WIN_ROWS)], sem_s1)
    d_rp = pltpu.make_async_copy(qp_sh.at[pl.ds(row0, WIN_ROWS)],
                                 qp_w.at[pl.ds(0, WIN_ROWS)], sem_s2)
    d_rn = pltpu.make_async_copy(qn_sh.at[pl.ds(row0, WIN_ROWS)],
                                 qn_w.at[pl.ds(0, WIN_ROWS)], sem_s3)
    d_rw.start()
    d_rp.start()
    d_rn.start()

    @pl.when(on_core0)
    def _p3():
        pltpu.sync_copy(pres_sh.at[pl.ds(nbase, NODES_PER_TILE)], pres_v)
        acc = jnp.zeros((16,), _f32)
        for j in range(NODES_PER_TILE // 16):
            p = pres_v[pl.ds(j * 16, 16)]
            bt = bit_v[pl.ds(nbase + j * 16, 16)]
            acc = acc + p.astype(_f32) * bt
        acc_v[...] = acc
        pltpu.sync_copy(acc_v, psum_sh.at[iota_v], add=True)

    d_rw.wait()
    d_rp.wait()
    d_rn.wait()

    def compute(i, p):
        @pl.loop(0, CHUNK_ROWS // 8)
        def _grp(g):
            w0 = i * CHUNK_ROWS + g * 8
            v_sc = sc_w[pl.ds(w0, 16)]
            v_qp = qp_w[pl.ds(w0, 16)]
            v_qn = qn_w[pl.ds(w0, 16)]
            v_inv = 1.0 / v_sc
            for j in range(8):
                r = g * 8 + j
                sc_b = jnp.broadcast_to(v_sc[j], (16,))
                inv_b = jnp.broadcast_to(v_inv[j], (16,))
                qp_b = jnp.broadcast_to(v_qp[j], (16,))
                qn_b = jnp.broadcast_to(v_qn[j], (16,))
                for k in range(D_FEAT // 16):
                    x = ins[p][r, pl.ds(k * 16, 16)]
                    q = x * inv_b
                    qc = jnp.minimum(jnp.maximum(q, qn_b), qp_b)
                    qr = (qc + MAGIC) - MAGIC
                    outs[p][r, pl.ds(k * 16, 16)] = qr * sc_b

    @pl.loop(0, MAX_CH // 2)
    def _pair(t):
        iA = 2 * t
        wait_in(0)

        @pl.when(t > 0)
        def _():
            wait_out(0)

        compute(iA, 0)

        @pl.when(iA + 2 < nch)
        def _():
            start_in(iA + 2, 0)

        start_out(iA, 0)

        iB = iA + 1

        @pl.when(iB < nch)
        def _chunk_b():
            wait_in(1)

            @pl.when(t > 0)
            def _():
                wait_out(1)

            compute(iB, 1)

            @pl.when(iB + 2 < nch)
            def _():
                start_in(iB + 2, 1)

            start_out(iB, 1)

    wait_out(0)
    wait_out(1)

    plsc.subcore_barrier()

    @pl.when(on_core0 & (s == 0))
    def _p4():
        pltpu.sync_copy(psum_sh, acc_v)
        tot = jnp.sum(acc_v[...])
        val = tot * (float(D_FEAT) / 8.0 / 1024.0)
        acc_v[...] = jnp.broadcast_to(val, (16,))
        pltpu.sync_copy(acc_v.at[pl.ds(0, 1)], bs_o)


_sc_mesh = plsc.VectorSubcoreMesh(core_axis_name="c", subcore_axis_name="s")

_sc_params = pltpu.CompilerParams()
if "needs_layout_passes" in pltpu.CompilerParams.__dataclass_fields__:
    _sc_params = dataclasses.replace(_sc_params, needs_layout_passes=False)

_sc_call = functools.partial(
    pl.kernel,
    compiler_params=_sc_params,
    out_type=(
        jax.ShapeDtypeStruct((N_NODES, D_FEAT), _f32),
        jax.ShapeDtypeStruct((1,), _f32),
    ),
    mesh=_sc_mesh,
    scratch_types=[
        pltpu.VMEM_SHARED((N_PAD,), _i32),
        pltpu.VMEM_SHARED((N_PAD,), _i32),
        pltpu.VMEM_SHARED((16,), _f32),
        pltpu.VMEM_SHARED((N_PAD,), _f32),
        pltpu.VMEM_SHARED((N_PAD,), _f32),
        pltpu.VMEM_SHARED((N_PAD,), _f32),
        pltpu.VMEM((EDGES_PER_TILE,), _i32),
        pltpu.VMEM((EDGES_PER_TILE,), _i32),
        pltpu.VMEM((N_PAD,), _f32),
        pltpu.VMEM((N_PAD,), _f32),
        pltpu.VMEM((NODES_PER_TILE,), _i32),
        pltpu.VMEM((NODES_PER_TILE,), _i32),
        pltpu.VMEM((NODES_PER_TILE,), _f32),
        pltpu.VMEM((NODES_PER_TILE,), _f32),
        pltpu.VMEM((NODES_PER_TILE,), _f32),
        pltpu.VMEM((NODES_PER_TILE,), _i32),
        pltpu.VMEM((NODES_PER_TILE,), _i32),
        pltpu.VMEM((16,), _i32),
        pltpu.VMEM((16,), _f32),
        pltpu.VMEM((NODES_PER_TILE,), _i32),
        pltpu.VMEM((16,), _f32),
        pltpu.VMEM((WIN_ROWS + 16,), _f32),
        pltpu.VMEM((WIN_ROWS + 16,), _f32),
        pltpu.VMEM((WIN_ROWS + 16,), _f32),
        pltpu.VMEM((CHUNK_ROWS, D_FEAT), _f32),
        pltpu.VMEM((CHUNK_ROWS, D_FEAT), _f32),
        pltpu.VMEM((CHUNK_ROWS, D_FEAT), _f32),
        pltpu.VMEM((CHUNK_ROWS, D_FEAT), _f32),
        pltpu.SemaphoreType.DMA,
        pltpu.SemaphoreType.DMA,
        pltpu.SemaphoreType.DMA,
        pltpu.SemaphoreType.DMA,
        pltpu.SemaphoreType.DMA,
        pltpu.SemaphoreType.DMA,
        pltpu.SemaphoreType.DMA,
        pltpu.SemaphoreType.DMA,
        pltpu.SemaphoreType.DMA,
    ],
)(_sc_body)


def kernel(fea, edge_index, gama, bit):
    fea_q, bs = _sc_call(
        edge_index.reshape(-1), gama.reshape(-1), bit.reshape(-1), fea)
    return fea_q, bs.reshape(())
